# baseline (speedup 1.0000x reference)
import functools

import jax
import jax.numpy as jnp
from jax import lax
from jax.experimental import pallas as pl
from jax.experimental.pallas import tpu as pltpu
from jax.experimental.pallas import tpu_sc as plsc

_N_WT = 64
_L = 512
_SR = 16000
_B = 8
_T = 64000
_BLOCK = 160
_NBLK = _T // _BLOCK
_ROWS = 500
_LANES = 128

_NW = 32
_WBLK = 13
_WSAMP = _WBLK * _BLOCK


def _prep_body(pitch_ref, wt_ref, att_ref, gmap_ref, gidx_ref, alpha_ref,
               m_ref, d_ref):
    w = wt_ref[...]
    w = jnp.concatenate([w[:4], jnp.tanh(w[4:])], axis=0)
    a = att_ref[...]
    a = a - jnp.max(a, axis=0, keepdims=True)
    e = jnp.exp(a)
    att = e / jnp.sum(e, axis=0, keepdims=True)
    m = lax.dot_general(att, w, (((0,), (0,)), ((), ())),
                        preferred_element_type=jnp.float32,
                        precision=lax.Precision.HIGHEST)
    m_ref[...] = m
    d_ref[...] = jnp.concatenate([m[:, 1:], m[:, :1]], axis=1) - m

    inc2 = pitch_ref[...] / jnp.float32(_SR) * jnp.float32(_L)
    ui = lax.broadcasted_iota(jnp.int32, (_LANES, _LANES), 0)
    uj = lax.broadcasted_iota(jnp.int32, (_LANES, _LANES), 1)
    u128 = (ui <= uj).astype(jnp.float32)
    y2 = lax.dot_general(inc2, u128, (((1,), (0,)), ((), ())),
                         preferred_element_type=jnp.float32,
                         precision=lax.Precision.HIGHEST)
    y3 = y2.reshape(_B, _ROWS, _LANES)
    inc3 = inc2.reshape(_B, _ROWS, _LANES)
    rows = y3[:, :, _LANES - 1]
    rows = rows - jnp.float32(_L) * jnp.floor(rows * jnp.float32(1.0 / _L))
    hi = jnp.floor(rows)
    fr = rows - hi
    si = lax.broadcasted_iota(jnp.int32, (_ROWS, _ROWS), 0)
    sj = lax.broadcasted_iota(jnp.int32, (_ROWS, _ROWS), 1)
    su = (si < sj).astype(jnp.float32)
    exhi = lax.dot_general(hi, su, (((1,), (0,)), ((), ())),
                           preferred_element_type=jnp.float32,
                           precision=lax.Precision.HIGHEST)
    exfr = lax.dot_general(fr, su, (((1,), (0,)), ((), ())),
                           preferred_element_type=jnp.float32,
                           precision=lax.Precision.HIGHEST)
    exhi = exhi - jnp.float32(_L) * jnp.floor(exhi * jnp.float32(1.0 / _L))
    ex = exhi + exfr
    idx = y3 + ex[:, :, None] - inc3[0:1]
    ph = idx - jnp.float32(_L) * jnp.floor(idx * jnp.float32(1.0 / _L))
    ph = jnp.where(ph >= jnp.float32(_L), ph - jnp.float32(_L), ph)
    lo = jnp.floor(ph)
    alpha_ref[...] = ph - lo
    loi = jnp.minimum(lo.astype(jnp.int32), _L - 1)
    gidx_ref[...] = gmap_ref[...] + loi


def _sc_body(g_hbm, al_hbm, amp_hbm, m_hbm, d_hbm, out_hbm,
             g_v, al_v, amp_v, out_v, mt_v, dt_v, sem):
    cid = lax.axis_index("c")
    sid = lax.axis_index("s")
    wid = sid * 2 + cid
    blk0 = (wid * 25) // 2
    t0 = blk0 * _BLOCK

    cp = pltpu.make_async_copy
    dmas = [
        cp(m_hbm.at[pl.ds(blk0 * _L, _WBLK * _L)], mt_v, sem),
        cp(d_hbm.at[pl.ds(blk0 * _L, _WBLK * _L)], dt_v, sem),
    ]
    for b in range(_B):
        src = pl.ds(b * _T + t0, _WSAMP)
        dst = pl.ds(b * _WSAMP, _WSAMP)
        dmas.append(cp(g_hbm.at[src], g_v.at[dst], sem))
        dmas.append(cp(al_hbm.at[src], al_v.at[dst], sem))
        dmas.append(cp(amp_hbm.at[src], amp_v.at[dst], sem))
    for dma in dmas:
        dma.start()
    for dma in dmas:
        dma.wait()

    gbase = blk0 * _L

    def body(i, carry):
        off = i * 16
        g = g_v[pl.ds(off, 16)] - gbase
        alpha = al_v[pl.ds(off, 16)]
        amp = amp_v[pl.ds(off, 16)]
        mval = plsc.load_gather(mt_v, [g])
        dval = plsc.load_gather(dt_v, [g])
        out_v[pl.ds(off, 16)] = amp * (mval + alpha * dval)
        return carry

    lax.fori_loop(0, _B * _WSAMP // 16, body, 0)

    odmas = [cp(out_v.at[pl.ds(b * _WSAMP, _WSAMP)],
                out_hbm.at[pl.ds(b * _T + t0, _WSAMP)], sem)
             for b in range(_B)]
    for dma in odmas:
        dma.start()
    for dma in odmas:
        dma.wait()


def kernel(pitch, amplitude, wavetables, attention):
    pitch2 = pitch.reshape(_B * _ROWS, _LANES)
    gmap = ((jnp.arange(_T, dtype=jnp.int32) // _BLOCK) * _L).reshape(
        _ROWS, _LANES)[None]
    gidx, alpha, m, d = pl.pallas_call(
        _prep_body,
        out_shape=(
            jax.ShapeDtypeStruct((_B, _ROWS, _LANES), jnp.int32),
            jax.ShapeDtypeStruct((_B, _ROWS, _LANES), jnp.float32),
            jax.ShapeDtypeStruct((_NBLK, _L), jnp.float32),
            jax.ShapeDtypeStruct((_NBLK, _L), jnp.float32),
        ),
    )(pitch2, wavetables, attention, gmap)

    mesh = plsc.VectorSubcoreMesh(core_axis_name="c", subcore_axis_name="s")
    sc = functools.partial(
        pl.kernel,
        mesh=mesh,
        compiler_params=pltpu.CompilerParams(needs_layout_passes=False),
        out_type=jax.ShapeDtypeStruct((_B * _T,), jnp.float32),
        scratch_types=[
            pltpu.VMEM((_B * _WSAMP,), jnp.int32),
            pltpu.VMEM((_B * _WSAMP,), jnp.float32),
            pltpu.VMEM((_B * _WSAMP,), jnp.float32),
            pltpu.VMEM((_B * _WSAMP,), jnp.float32),
            pltpu.VMEM((_WBLK * _L,), jnp.float32),
            pltpu.VMEM((_WBLK * _L,), jnp.float32),
            pltpu.SemaphoreType.DMA,
        ],
    )(_sc_body)
    out = sc(gidx.reshape(_B * _T), alpha.reshape(_B * _T),
             amplitude.reshape(_B * _T), m.reshape(_NBLK * _L),
             d.reshape(_NBLK * _L))
    return out.reshape(_B, _T, 1)


# speedup vs baseline: 1.1788x; 1.1788x over previous
import functools

import jax
import jax.numpy as jnp
from jax import lax
from jax.experimental import pallas as pl
from jax.experimental.pallas import tpu as pltpu
from jax.experimental.pallas import tpu_sc as plsc

_N_WT = 64
_L = 512
_SR = 16000
_B = 8
_T = 64000
_BLOCK = 160
_NBLK = _T // _BLOCK
_ROWS = 500
_LANES = 128

_NW = 32
_WBLK = 13
_WSAMP = _WBLK * _BLOCK
_ABITS = 13
_ASCALE = 1 << _ABITS


def _prep_body(pitch_ref, wt_ref, att_ref, gmap_ref, pk_ref, m_ref, d_ref):
    w = wt_ref[...]
    w = jnp.concatenate([w[:4], jnp.tanh(w[4:])], axis=0)
    a = att_ref[...]
    a = a - jnp.max(a, axis=0, keepdims=True)
    e = jnp.exp(a)
    att = e / jnp.sum(e, axis=0, keepdims=True)
    m = lax.dot_general(att, w, (((0,), (0,)), ((), ())),
                        preferred_element_type=jnp.float32,
                        precision=lax.Precision.HIGHEST)
    m_ref[...] = m
    d_ref[...] = jnp.concatenate([m[:, 1:], m[:, :1]], axis=1) - m

    inc2 = pitch_ref[...] / jnp.float32(_SR) * jnp.float32(_L)
    ui = lax.broadcasted_iota(jnp.int32, (_LANES, _LANES), 0)
    uj = lax.broadcasted_iota(jnp.int32, (_LANES, _LANES), 1)
    u128 = (ui <= uj).astype(jnp.float32)
    y2 = lax.dot_general(inc2, u128, (((1,), (0,)), ((), ())),
                         preferred_element_type=jnp.float32,
                         precision=lax.Precision.HIGHEST)
    y3 = y2.reshape(_B, _ROWS, _LANES)
    inc3 = inc2.reshape(_B, _ROWS, _LANES)
    rows = y3[:, :, _LANES - 1]
    rows = rows - jnp.float32(_L) * jnp.floor(rows * jnp.float32(1.0 / _L))
    hi = jnp.floor(rows)
    fr = rows - hi
    si = lax.broadcasted_iota(jnp.int32, (_ROWS, _ROWS), 0)
    sj = lax.broadcasted_iota(jnp.int32, (_ROWS, _ROWS), 1)
    su = (si < sj).astype(jnp.float32)
    exhi = lax.dot_general(hi, su, (((1,), (0,)), ((), ())),
                           preferred_element_type=jnp.float32,
                           precision=lax.Precision.HIGHEST)
    exfr = lax.dot_general(fr, su, (((1,), (0,)), ((), ())),
                           preferred_element_type=jnp.float32,
                           precision=lax.Precision.HIGHEST)
    exhi = exhi - jnp.float32(_L) * jnp.floor(exhi * jnp.float32(1.0 / _L))
    ex = exhi + exfr
    idx = y3 + ex[:, :, None] - inc3[0:1]
    ph = idx - jnp.float32(_L) * jnp.floor(idx * jnp.float32(1.0 / _L))
    ph = jnp.where(ph >= jnp.float32(_L), ph - jnp.float32(_L), ph)
    lo = jnp.floor(ph)
    ai = ((ph - lo) * jnp.float32(_ASCALE) + jnp.float32(0.5)).astype(jnp.int32)
    ai = jnp.minimum(ai, _ASCALE - 1)
    loi = jnp.minimum(lo.astype(jnp.int32), _L - 1)
    g = gmap_ref[...] + loi
    pk_ref[...] = lax.shift_left(g, _ABITS) + ai


def _sc_body(pk_hbm, amp_hbm, m_hbm, d_hbm, out_hbm,
             pk_v, amp_v, out_v, mt_v, dt_v, sem_a, sem_b):
    cid = lax.axis_index("c")
    sid = lax.axis_index("s")
    wid = sid * 2 + cid
    blk0 = (wid * 25) // 2
    t0 = blk0 * _BLOCK

    cp = pltpu.make_async_copy
    half = _B // 2
    dmas_a = [
        cp(m_hbm.at[pl.ds(blk0 * _L, _WBLK * _L)], mt_v, sem_a),
        cp(d_hbm.at[pl.ds(blk0 * _L, _WBLK * _L)], dt_v, sem_a),
    ]
    dmas_b = []
    for b in range(_B):
        src = pl.ds(b * _T + t0, _WSAMP)
        dst = pl.ds(b * _WSAMP, _WSAMP)
        sem = sem_a if b < half else sem_b
        lst = dmas_a if b < half else dmas_b
        lst.append(cp(pk_hbm.at[src], pk_v.at[dst], sem))
        lst.append(cp(amp_hbm.at[src], amp_v.at[dst], sem))
    for dma in dmas_a + dmas_b:
        dma.start()
    for dma in dmas_a:
        dma.wait()

    gshift = lax.shift_left(blk0 * _L, _ABITS)
    inv = jnp.float32(1.0 / _ASCALE)
    nhalf = half * _WSAMP // 16

    def make_body(base):
        def body(i):
            off = base + i * 16
            v = pk_v[pl.ds(off, 16)] - gshift
            g = lax.shift_right_logical(v, _ABITS)
            alpha = (v & (_ASCALE - 1)).astype(jnp.float32) * inv
            amp = amp_v[pl.ds(off, 16)]
            mval = plsc.load_gather(mt_v, [g])
            dval = plsc.load_gather(dt_v, [g])
            out_v[pl.ds(off, 16)] = amp * (mval + alpha * dval)
        return body

    plsc.parallel_loop(0, nhalf, 1, unroll=8)(make_body(0))

    odmas = [cp(out_v.at[pl.ds(b * _WSAMP, _WSAMP)],
                out_hbm.at[pl.ds(b * _T + t0, _WSAMP)], sem_a)
             for b in range(half)]
    for dma in odmas:
        dma.start()

    for dma in dmas_b:
        dma.wait()
    plsc.parallel_loop(0, nhalf, 1, unroll=8)(make_body(half * _WSAMP))

    odmas2 = [cp(out_v.at[pl.ds(b * _WSAMP, _WSAMP)],
                 out_hbm.at[pl.ds(b * _T + t0, _WSAMP)], sem_b)
              for b in range(half, _B)]
    for dma in odmas2:
        dma.start()
    for dma in odmas + odmas2:
        dma.wait()


def kernel(pitch, amplitude, wavetables, attention):
    pitch2 = pitch.reshape(_B * _ROWS, _LANES)
    gmap = ((jnp.arange(_T, dtype=jnp.int32) // _BLOCK) * _L).reshape(
        _ROWS, _LANES)[None]
    pk, m, d = pl.pallas_call(
        _prep_body,
        out_shape=(
            jax.ShapeDtypeStruct((_B, _ROWS, _LANES), jnp.int32),
            jax.ShapeDtypeStruct((_NBLK, _L), jnp.float32),
            jax.ShapeDtypeStruct((_NBLK, _L), jnp.float32),
        ),
    )(pitch2, wavetables, attention, gmap)

    mesh = plsc.VectorSubcoreMesh(core_axis_name="c", subcore_axis_name="s")
    sc = functools.partial(
        pl.kernel,
        mesh=mesh,
        compiler_params=pltpu.CompilerParams(needs_layout_passes=False),
        out_type=jax.ShapeDtypeStruct((_B * _T,), jnp.float32),
        scratch_types=[
            pltpu.VMEM((_B * _WSAMP,), jnp.int32),
            pltpu.VMEM((_B * _WSAMP,), jnp.float32),
            pltpu.VMEM((_B * _WSAMP,), jnp.float32),
            pltpu.VMEM((_WBLK * _L,), jnp.float32),
            pltpu.VMEM((_WBLK * _L,), jnp.float32),
            pltpu.SemaphoreType.DMA,
            pltpu.SemaphoreType.DMA,
        ],
    )(_sc_body)
    out = sc(pk.reshape(_B * _T), amplitude.reshape(_B * _T),
             m.reshape(_NBLK * _L), d.reshape(_NBLK * _L))
    return out.reshape(_B, _T, 1)


# speedup vs baseline: 1.1905x; 1.0099x over previous
import functools

import jax
import jax.numpy as jnp
from jax import lax
from jax.experimental import pallas as pl
from jax.experimental.pallas import tpu as pltpu
from jax.experimental.pallas import tpu_sc as plsc

_N_WT = 64
_L = 512
_SR = 16000
_B = 8
_T = 64000
_BLOCK = 160
_NBLK = _T // _BLOCK
_ROWS = 500
_LANES = 128

_NW = 32
_WBLK = 13
_WSAMP = _WBLK * _BLOCK
_ABITS = 13
_ASCALE = 1 << _ABITS


def _prep_body(pitch_ref, wt_ref, att_ref, gmap_ref, u128_ref, su_ref,
               pk_ref, m_ref, d_ref):
    w = wt_ref[...]
    w = jnp.concatenate([w[:4], jnp.tanh(w[4:])], axis=0)
    a = att_ref[...]
    a = a - jnp.max(a, axis=0, keepdims=True)
    e = jnp.exp(a)
    att = e / jnp.sum(e, axis=0, keepdims=True)
    m = lax.dot_general(att, w, (((0,), (0,)), ((), ())),
                        preferred_element_type=jnp.float32,
                        precision=lax.Precision.HIGHEST)
    m_ref[...] = m
    d_ref[...] = jnp.concatenate([m[:, 1:], m[:, :1]], axis=1) - m

    inc2 = pitch_ref[...] * jnp.float32(float(_L) / _SR)
    y2 = lax.dot_general(inc2, u128_ref[...], (((1,), (0,)), ((), ())),
                         preferred_element_type=jnp.float32,
                         precision=lax.Precision.HIGHEST)
    y3 = y2.reshape(_B, _ROWS, _LANES)
    inc3 = inc2.reshape(_B, _ROWS, _LANES)
    rows = jnp.sum(inc3, axis=2)
    rows = rows - jnp.float32(_L) * jnp.floor(rows * jnp.float32(1.0 / _L))
    hi = jnp.floor(rows)
    fr = rows - hi
    su = su_ref[...]
    exhi = lax.dot_general(hi, su, (((1,), (0,)), ((), ())),
                           preferred_element_type=jnp.float32,
                           precision=lax.Precision.HIGHEST)
    exfr = lax.dot_general(fr, su, (((1,), (0,)), ((), ())),
                           preferred_element_type=jnp.float32,
                           precision=lax.Precision.HIGHEST)
    exhi = exhi - jnp.float32(_L) * jnp.floor(exhi * jnp.float32(1.0 / _L))
    ex = exhi + exfr
    idx = y3 + ex[:, :, None] - inc3[0:1]
    ph = idx - jnp.float32(_L) * jnp.floor(idx * jnp.float32(1.0 / _L))
    pki = (ph * jnp.float32(_ASCALE)).astype(jnp.int32)
    pki = jnp.minimum(pki, _L * _ASCALE - 1)
    pk_ref[...] = gmap_ref[...] + pki


def _sc_body(pk_hbm, amp_hbm, m_hbm, d_hbm, out_hbm,
             pk_v, amp_v, out_v, mt_v, dt_v, sem_a, sem_b):
    cid = lax.axis_index("c")
    sid = lax.axis_index("s")
    wid = sid * 2 + cid
    blk0 = (wid * 25) // 2
    t0 = blk0 * _BLOCK

    cp = pltpu.make_async_copy
    half = _B // 2
    dmas_a = [
        cp(m_hbm.at[pl.ds(blk0 * _L, _WBLK * _L)], mt_v, sem_a),
        cp(d_hbm.at[pl.ds(blk0 * _L, _WBLK * _L)], dt_v, sem_a),
    ]
    dmas_b = []
    for b in range(_B):
        src = pl.ds(b * _T + t0, _WSAMP)
        dst = pl.ds(b * _WSAMP, _WSAMP)
        sem = sem_a if b < half else sem_b
        lst = dmas_a if b < half else dmas_b
        lst.append(cp(pk_hbm.at[src], pk_v.at[dst], sem))
        lst.append(cp(amp_hbm.at[src], amp_v.at[dst], sem))
    for dma in dmas_a + dmas_b:
        dma.start()
    for dma in dmas_a:
        dma.wait()

    gshift = lax.shift_left(blk0 * _L, _ABITS)
    inv = jnp.float32(1.0 / _ASCALE)
    nhalf = half * _WSAMP // 16

    def make_body(base):
        def body(i):
            off = base + i * 16
            v = pk_v[pl.ds(off, 16)] - gshift
            g = lax.shift_right_logical(v, _ABITS)
            alpha = (v & (_ASCALE - 1)).astype(jnp.float32) * inv
            amp = amp_v[pl.ds(off, 16)]
            mval = plsc.load_gather(mt_v, [g])
            dval = plsc.load_gather(dt_v, [g])
            out_v[pl.ds(off, 16)] = amp * (mval + alpha * dval)
        return body

    plsc.parallel_loop(0, nhalf, 1, unroll=8)(make_body(0))

    odmas = [cp(out_v.at[pl.ds(b * _WSAMP, _WSAMP)],
                out_hbm.at[pl.ds(b * _T + t0, _WSAMP)], sem_a)
             for b in range(half)]
    for dma in odmas:
        dma.start()

    for dma in dmas_b:
        dma.wait()
    plsc.parallel_loop(0, nhalf, 1, unroll=8)(make_body(half * _WSAMP))

    odmas2 = [cp(out_v.at[pl.ds(b * _WSAMP, _WSAMP)],
                 out_hbm.at[pl.ds(b * _T + t0, _WSAMP)], sem_b)
              for b in range(half, _B)]
    for dma in odmas2:
        dma.start()
    for dma in odmas + odmas2:
        dma.wait()


def kernel(pitch, amplitude, wavetables, attention):
    pitch2 = pitch.reshape(_B * _ROWS, _LANES)
    gmap = (((jnp.arange(_T, dtype=jnp.int32) // _BLOCK) * _L) << _ABITS
            ).reshape(_ROWS, _LANES)[None]
    u128 = jnp.triu(jnp.ones((_LANES, _LANES), jnp.float32))
    su = jnp.triu(jnp.ones((_ROWS, _ROWS), jnp.float32), k=1)
    pk, m, d = pl.pallas_call(
        _prep_body,
        out_shape=(
            jax.ShapeDtypeStruct((_B, _ROWS, _LANES), jnp.int32),
            jax.ShapeDtypeStruct((_NBLK, _L), jnp.float32),
            jax.ShapeDtypeStruct((_NBLK, _L), jnp.float32),
        ),
    )(pitch2, wavetables, attention, gmap, u128, su)

    mesh = plsc.VectorSubcoreMesh(core_axis_name="c", subcore_axis_name="s")
    sc = functools.partial(
        pl.kernel,
        mesh=mesh,
        compiler_params=pltpu.CompilerParams(needs_layout_passes=False),
        out_type=jax.ShapeDtypeStruct((_B * _T,), jnp.float32),
        scratch_types=[
            pltpu.VMEM((_B * _WSAMP,), jnp.int32),
            pltpu.VMEM((_B * _WSAMP,), jnp.float32),
            pltpu.VMEM((_B * _WSAMP,), jnp.float32),
            pltpu.VMEM((_WBLK * _L,), jnp.float32),
            pltpu.VMEM((_WBLK * _L,), jnp.float32),
            pltpu.SemaphoreType.DMA,
            pltpu.SemaphoreType.DMA,
        ],
    )(_sc_body)
    out = sc(pk.reshape(_B * _T), amplitude.reshape(_B * _T),
             m.reshape(_NBLK * _L), d.reshape(_NBLK * _L))
    return out.reshape(_B, _T, 1)


# speedup vs baseline: 1.2227x; 1.0270x over previous
import functools

import jax
import jax.numpy as jnp
from jax import lax
from jax.experimental import pallas as pl
from jax.experimental.pallas import tpu as pltpu
from jax.experimental.pallas import tpu_sc as plsc

_N_WT = 64
_L = 512
_SR = 16000
_B = 8
_T = 64000
_BLOCK = 160
_NBLK = _T // _BLOCK
_ROWS = 500
_LANES = 128

_NW = 32
_WBLK = 13
_WSAMP = _WBLK * _BLOCK
_ABITS = 13
_ASCALE = 1 << _ABITS
_STRIDE = 520


def _prep_body(pitch_ref, wt_ref, att_ref, gmap_ref, u128_ref, su_ref,
               pk_ref, m_ref):
    w = wt_ref[...]
    w = jnp.concatenate([w[:4], jnp.tanh(w[4:])], axis=0)
    a = att_ref[...]
    a = a - jnp.max(a, axis=0, keepdims=True)
    e = jnp.exp(a)
    att = e / jnp.sum(e, axis=0, keepdims=True)
    m = lax.dot_general(att, w, (((0,), (0,)), ((), ())),
                        preferred_element_type=jnp.float32,
                        precision=lax.Precision.HIGHEST)
    m_ref[...] = jnp.concatenate([m, m[:, :_STRIDE - _L]], axis=1)

    inc2 = pitch_ref[...] * jnp.float32(float(_L) / _SR)
    y2 = lax.dot_general(inc2, u128_ref[...], (((1,), (0,)), ((), ())),
                         preferred_element_type=jnp.float32,
                         precision=lax.Precision.HIGHEST)
    y3 = y2.reshape(_B, _ROWS, _LANES)
    inc3 = inc2.reshape(_B, _ROWS, _LANES)
    rows = jnp.sum(inc3, axis=2)
    rows = rows - jnp.float32(_L) * jnp.floor(rows * jnp.float32(1.0 / _L))
    hi = jnp.floor(rows)
    fr = rows - hi
    su = su_ref[...]
    exhi = lax.dot_general(hi, su, (((1,), (0,)), ((), ())),
                           preferred_element_type=jnp.float32,
                           precision=lax.Precision.HIGHEST)
    exfr = lax.dot_general(fr, su, (((1,), (0,)), ((), ())),
                           preferred_element_type=jnp.float32,
                           precision=lax.Precision.HIGHEST)
    exhi = exhi - jnp.float32(_L) * jnp.floor(exhi * jnp.float32(1.0 / _L))
    ex = exhi + exfr
    idx = y3 + ex[:, :, None] - inc3[0:1]
    ph = idx - jnp.float32(_L) * jnp.floor(idx * jnp.float32(1.0 / _L))
    pki = (ph * jnp.float32(_ASCALE)).astype(jnp.int32)
    pki = jnp.minimum(pki, _L * _ASCALE - 1)
    pk_ref[...] = gmap_ref[...] + pki


def _sc_body(pk_hbm, amp_hbm, m_hbm, out_hbm,
             pk_v, amp_v, out_v, mt_v, sem_a, sem_b, sem_o):
    cid = lax.axis_index("c")
    sid = lax.axis_index("s")
    wid = sid * 2 + cid
    blk0 = (wid * 25) // 2
    t0 = blk0 * _BLOCK

    cp = pltpu.make_async_copy
    sems = (sem_a, sem_b)
    npair = _B // 2
    in_dmas = [[cp(m_hbm.at[pl.ds(blk0 * _STRIDE, _WBLK * _STRIDE)],
                   mt_v, sem_a)]]
    for p in range(npair):
        lst = [] if p else in_dmas[0]
        for b in (2 * p, 2 * p + 1):
            src = pl.ds(b * _T + t0, _WSAMP)
            dst = pl.ds(b * _WSAMP, _WSAMP)
            lst.append(cp(pk_hbm.at[src], pk_v.at[dst], sems[p % 2]))
            lst.append(cp(amp_hbm.at[src], amp_v.at[dst], sems[p % 2]))
        if p:
            in_dmas.append(lst)
    for lst in in_dmas:
        for dma in lst:
            dma.start()

    gshift = lax.shift_left(blk0 * _STRIDE, _ABITS)
    inv = jnp.float32(1.0 / _ASCALE)
    npp = 2 * _WSAMP // 16

    def make_body(base):
        def body(i):
            off = base + i * 16
            v = pk_v[pl.ds(off, 16)] - gshift
            g = lax.shift_right_logical(v, _ABITS)
            alpha = (v & (_ASCALE - 1)).astype(jnp.float32) * inv
            amp = amp_v[pl.ds(off, 16)]
            mval = plsc.load_gather(mt_v, [g])
            hval = plsc.load_gather(mt_v, [g + 1])
            out_v[pl.ds(off, 16)] = amp * (mval + alpha * (hval - mval))
        return body

    out_dmas = []
    for p in range(npair):
        for dma in in_dmas[p]:
            dma.wait()
        plsc.parallel_loop(0, npp, 1, unroll=8)(make_body(2 * p * _WSAMP))
        for b in (2 * p, 2 * p + 1):
            od = cp(out_v.at[pl.ds(b * _WSAMP, _WSAMP)],
                    out_hbm.at[pl.ds(b * _T + t0, _WSAMP)], sem_o)
            od.start()
            out_dmas.append(od)
    for dma in out_dmas:
        dma.wait()


def kernel(pitch, amplitude, wavetables, attention):
    pitch2 = pitch.reshape(_B * _ROWS, _LANES)
    gmap = (((jnp.arange(_T, dtype=jnp.int32) // _BLOCK) * _STRIDE) << _ABITS
            ).reshape(_ROWS, _LANES)[None]
    u128 = jnp.triu(jnp.ones((_LANES, _LANES), jnp.float32))
    su = jnp.triu(jnp.ones((_ROWS, _ROWS), jnp.float32), k=1)
    pk, m = pl.pallas_call(
        _prep_body,
        out_shape=(
            jax.ShapeDtypeStruct((_B, _ROWS, _LANES), jnp.int32),
            jax.ShapeDtypeStruct((_NBLK, _STRIDE), jnp.float32),
        ),
    )(pitch2, wavetables, attention, gmap, u128, su)

    mesh = plsc.VectorSubcoreMesh(core_axis_name="c", subcore_axis_name="s")
    sc = functools.partial(
        pl.kernel,
        mesh=mesh,
        compiler_params=pltpu.CompilerParams(needs_layout_passes=False),
        out_type=jax.ShapeDtypeStruct((_B * _T,), jnp.float32),
        scratch_types=[
            pltpu.VMEM((_B * _WSAMP,), jnp.int32),
            pltpu.VMEM((_B * _WSAMP,), jnp.float32),
            pltpu.VMEM((_B * _WSAMP,), jnp.float32),
            pltpu.VMEM((_WBLK * _STRIDE,), jnp.float32),
            pltpu.SemaphoreType.DMA,
            pltpu.SemaphoreType.DMA,
            pltpu.SemaphoreType.DMA,
        ],
    )(_sc_body)
    out = sc(pk.reshape(_B * _T), amplitude.reshape(_B * _T),
             m.reshape(_NBLK * _STRIDE))
    return out.reshape(_B, _T, 1)


# speedup vs baseline: 1.2677x; 1.0368x over previous
import functools

import jax
import jax.numpy as jnp
from jax import lax
from jax.experimental import pallas as pl
from jax.experimental.pallas import tpu as pltpu
from jax.experimental.pallas import tpu_sc as plsc

_N_WT = 64
_L = 512
_SR = 16000
_B = 8
_T = 64000
_BLOCK = 160
_NBLK = _T // _BLOCK
_ROWS = 500
_LANES = 128

_NW = 32
_WBLK = 13
_WSAMP = _WBLK * _BLOCK
_ABITS = 13
_ASCALE = 1 << _ABITS
_STRIDE = 520


def _prep_body(pitch_ref, wt_ref, att_ref, gmap_ref, u128_ref, su_ref,
               pk_ref, m_ref):
    w = wt_ref[...]
    w = jnp.concatenate([w[:4], jnp.tanh(w[4:])], axis=0)
    a = att_ref[...]
    a = a - jnp.max(a, axis=0, keepdims=True)
    e = jnp.exp(a)
    att = e / jnp.sum(e, axis=0, keepdims=True)
    m = lax.dot_general(att, w, (((0,), (0,)), ((), ())),
                        preferred_element_type=jnp.float32,
                        precision=lax.Precision.HIGHEST)
    m_ref[...] = jnp.concatenate([m, m[:, :_STRIDE - _L]], axis=1)

    inc2 = pitch_ref[...] * jnp.float32(float(_L) / _SR)
    y2 = lax.dot_general(inc2, u128_ref[...], (((1,), (0,)), ((), ())),
                         preferred_element_type=jnp.float32,
                         precision=lax.Precision.HIGHEST)
    y3 = y2.reshape(_B, _ROWS, _LANES)
    inc3 = inc2.reshape(_B, _ROWS, _LANES)
    rows = jnp.sum(inc3, axis=2)
    rows = rows - jnp.float32(_L) * jnp.floor(rows * jnp.float32(1.0 / _L))
    hi = jnp.floor(rows)
    fr = rows - hi
    su = su_ref[...]
    exhi = lax.dot_general(hi, su, (((1,), (0,)), ((), ())),
                           preferred_element_type=jnp.float32,
                           precision=lax.Precision.HIGHEST)
    exfr = lax.dot_general(fr, su, (((1,), (0,)), ((), ())),
                           preferred_element_type=jnp.float32,
                           precision=lax.Precision.HIGHEST)
    exhi = exhi - jnp.float32(_L) * jnp.floor(exhi * jnp.float32(1.0 / _L))
    ex = exhi + exfr
    idx = y3 + ex[:, :, None] - inc3[0:1]
    ph = idx - jnp.float32(_L) * jnp.floor(idx * jnp.float32(1.0 / _L))
    pki = (ph * jnp.float32(_ASCALE)).astype(jnp.int32)
    pki = jnp.minimum(pki, _L * _ASCALE - 1)
    pk = gmap_ref[...] + pki
    pk_ref[...] = pk.reshape(_B * _ROWS, _LANES)


def _sc_body(pk_hbm, amp_hbm, m_hbm, out_hbm,
             pk_v, amp_v, out_v, mt_v, sem_a, sem_b, sem_o):
    cid = lax.axis_index("c")
    sid = lax.axis_index("s")
    wid = sid * 2 + cid
    blk0 = (wid * 25) // 2
    t0 = blk0 * _BLOCK

    cp = pltpu.make_async_copy
    sems = (sem_a, sem_b)
    npair = _B // 2
    in_dmas = [[cp(m_hbm.at[pl.ds(blk0 * _STRIDE, _WBLK * _STRIDE)],
                   mt_v, sem_a)]]
    for p in range(npair):
        lst = [] if p else in_dmas[0]
        for b in (2 * p, 2 * p + 1):
            src = pl.ds(b * _T + t0, _WSAMP)
            dst = pl.ds(b * _WSAMP, _WSAMP)
            lst.append(cp(pk_hbm.at[src], pk_v.at[dst], sems[p % 2]))
            lst.append(cp(amp_hbm.at[src], amp_v.at[dst], sems[p % 2]))
        if p:
            in_dmas.append(lst)
    for lst in in_dmas:
        for dma in lst:
            dma.start()

    gshift = lax.shift_left(blk0 * _STRIDE, _ABITS)
    inv = jnp.float32(1.0 / _ASCALE)
    npp = 2 * _WSAMP // 16

    def make_body(base):
        def body(i):
            off = base + i * 16
            v = pk_v[pl.ds(off, 16)] - gshift
            g = lax.shift_right_logical(v, _ABITS)
            alpha = (v & (_ASCALE - 1)).astype(jnp.float32) * inv
            amp = amp_v[pl.ds(off, 16)]
            mval = plsc.load_gather(mt_v, [g])
            hval = plsc.load_gather(mt_v, [g + 1])
            out_v[pl.ds(off, 16)] = amp * (mval + alpha * (hval - mval))
        return body

    out_dmas = []
    for p in range(npair):
        for dma in in_dmas[p]:
            dma.wait()
        plsc.parallel_loop(0, npp, 1, unroll=8)(make_body(2 * p * _WSAMP))
        for b in (2 * p, 2 * p + 1):
            od = cp(out_v.at[pl.ds(b * _WSAMP, _WSAMP)],
                    out_hbm.at[pl.ds(b * _T + t0, _WSAMP)], sem_o)
            od.start()
            out_dmas.append(od)
    for dma in out_dmas:
        dma.wait()


def kernel(pitch, amplitude, wavetables, attention):
    pitch2 = pitch.reshape(_B * _ROWS, _LANES)
    gmap = (((jnp.arange(_T, dtype=jnp.int32) // _BLOCK) * _STRIDE) << _ABITS
            ).reshape(_ROWS, _LANES)[None]
    u128 = jnp.triu(jnp.ones((_LANES, _LANES), jnp.float32))
    su = jnp.triu(jnp.ones((_ROWS, _ROWS), jnp.float32), k=1)
    pk, m = pl.pallas_call(
        _prep_body,
        out_shape=(
            jax.ShapeDtypeStruct((_B * _ROWS, _LANES), jnp.int32),
            jax.ShapeDtypeStruct((_NBLK, _STRIDE), jnp.float32),
        ),
    )(pitch2, wavetables, attention, gmap, u128, su)

    mesh = plsc.VectorSubcoreMesh(core_axis_name="c", subcore_axis_name="s")
    sc = functools.partial(
        pl.kernel,
        mesh=mesh,
        compiler_params=pltpu.CompilerParams(needs_layout_passes=False),
        out_type=jax.ShapeDtypeStruct((_B * _T,), jnp.float32),
        scratch_types=[
            pltpu.VMEM((_B * _WSAMP,), jnp.int32),
            pltpu.VMEM((_B * _WSAMP,), jnp.float32),
            pltpu.VMEM((_B * _WSAMP,), jnp.float32),
            pltpu.VMEM((_WBLK * _STRIDE,), jnp.float32),
            pltpu.SemaphoreType.DMA,
            pltpu.SemaphoreType.DMA,
            pltpu.SemaphoreType.DMA,
        ],
    )(_sc_body)
    out = sc(pk.reshape(_B * _T), amplitude.reshape(_B * _T),
             m.reshape(_NBLK * _STRIDE))
    return out.reshape(_B, _T, 1)
